# Initial kernel scaffold; baseline (speedup 1.0000x reference)
import jax
import jax.numpy as jnp
from jax.experimental import pallas as pl


def kernel(x, weights, params, edge_index, batch):
    raise NotImplementedError("write your pallas kernel here")


# speedup vs baseline: 1.0566x; 1.0566x over previous
import functools

import jax
import jax.numpy as jnp
from jax import lax
from jax.experimental import pallas as pl
from jax.experimental.pallas import tpu as pltpu
from jax.experimental.pallas import tpu_sc as plsc

_N = 50000
_E = 800000
_G = 16
_H = 64

_NACC = 50048
_ROWS_PER_TILE = _NACC // 16
_EPAD = 819200
_IDX_ROWS = _EPAD // 128
_IDX_ROWS_PER_TILE = _IDX_ROWS // 16
_GROUPS = _IDX_ROWS_PER_TILE // 8


def _agg64_body(h0, h1, h2, h3, row2d, col2d, zeros, out0, out1, out2, out3,
                rbuf, cbuf, rows_v, acc, semg, sems):
    c = lax.axis_index("c")
    s = lax.axis_index("s")

    def do_quarter(h_hbm, out_hbm):
        sl = pl.ds(s * _ROWS_PER_TILE, _ROWS_PER_TILE)
        pltpu.sync_copy(zeros.at[sl], acc.at[sl])
        plsc.subcore_barrier()

        def body(g, carry):
            r0 = s * _IDX_ROWS_PER_TILE + g * 8
            pltpu.sync_copy(row2d.at[pl.ds(r0, 8)], rbuf)
            pltpu.sync_copy(col2d.at[pl.ds(r0, 8)], cbuf)
            gh = [pltpu.async_copy(h_hbm.at[rbuf.at[j]],
                                   rows_v.at[pl.ds(j * 128, 128)], semg)
                  for j in range(8)]
            for h in gh:
                h.wait()
            sh = [pltpu.async_copy(rows_v.at[pl.ds(j * 128, 128)],
                                   acc.at[cbuf.at[j]], sems, add=True)
                  for j in range(8)]
            for h in sh:
                h.wait()
            return carry

        lax.fori_loop(0, _GROUPS, body, 0)
        plsc.subcore_barrier()
        pltpu.sync_copy(acc.at[sl], out_hbm.at[sl])

    @pl.when(c == 0)
    def _():
        do_quarter(h0, out0)
        do_quarter(h1, out1)

    @pl.when(c == 1)
    def _():
        do_quarter(h2, out2)
        do_quarter(h3, out3)


@jax.jit
def _sc_agg64(h0, h1, h2, h3, row2d, col2d, zeros):
    mesh = plsc.VectorSubcoreMesh(core_axis_name="c", subcore_axis_name="s")
    k = functools.partial(
        pl.kernel,
        mesh=mesh,
        out_type=tuple(jax.ShapeDtypeStruct((_NACC, 16), jnp.float32)
                       for _ in range(4)),
        scratch_types=[
            pltpu.VMEM((8, 128), jnp.int32),
            pltpu.VMEM((8, 128), jnp.int32),
            pltpu.VMEM((1024, 16), jnp.float32),
            pltpu.VMEM_SHARED((_NACC, 16), jnp.float32),
            pltpu.SemaphoreType.DMA,
            pltpu.SemaphoreType.DMA,
        ],
        compiler_params=pltpu.CompilerParams(use_tc_tiling_on_sc=False),
    )(_agg64_body)
    return k(h0, h1, h2, h3, row2d, col2d, zeros)


def _seg_sum_feat(h, row2d, col2d, zeros):
    outs = _sc_agg64(h[:, :16], h[:, 16:32], h[:, 32:48], h[:, 48:],
                     row2d, col2d, zeros)
    return jnp.concatenate([o[:_N] for o in outs], axis=1)


def _bn(h, g, b):
    mean = jnp.mean(h, axis=0)
    var = jnp.var(h, axis=0)
    return (h - mean) / jnp.sqrt(var + 1e-5) * g + b


def _seg_sum_nodes(vals, col):
    return jax.ops.segment_sum(vals, col, num_segments=_N)


def kernel(x, weights, params, edge_index, batch):
    row, col = edge_index[0], edge_index[1]
    gnorm = float(_N) ** -0.5

    pad = _EPAD - _E
    row_p = jnp.concatenate([row, jnp.zeros((pad,), jnp.int32)])
    col_p = jnp.concatenate(
        [col, _N + (jnp.arange(pad, dtype=jnp.int32) % (_NACC - _N))])
    row2d = row_p.reshape(_IDX_ROWS, 128)
    col2d = col_p.reshape(_IDX_ROWS, 128)
    zeros = jnp.zeros((_NACC, 16), jnp.float32)

    ind0 = (jnp.abs(x) > 0).astype(jnp.float32)
    m = jnp.minimum(ind0 + _seg_sum_nodes(ind0[row], col), 1.0)
    masks = [m]
    for _ in range(2):
        m = jnp.minimum(m + _seg_sum_nodes(m[row], col), 1.0)
        masks.append(m)
    m1, m2, m3 = masks

    h = x[:, None]
    p = params["conv1"]
    agg = _seg_sum_nodes(x[row], col)[:, None]
    z = (1.0 + p["eps"]) * h + agg
    z = jax.nn.relu(z @ p["w1"] + p["b1"])
    z = jax.nn.relu(z @ p["w2"] + p["b2"])
    h = jax.nn.leaky_relu(_bn(z, p["bn_g"], p["bn_b"]))
    h = h * m1[:, None] * gnorm
    h = _bn(h, params["bn1"]["g"], params["bn1"]["b"])

    for p, bn, mk in zip(params["convs"], params["bns"], (m2, m3)):
        agg = _seg_sum_feat(h, row2d, col2d, zeros)
        z = (1.0 + p["eps"]) * h + agg
        z = jax.nn.relu(z @ p["w1"] + p["b1"])
        z = jax.nn.relu(z @ p["w2"] + p["b2"])
        z = _bn(z, p["bn_g"], p["bn_b"])
        h = h + jax.nn.leaky_relu(z)
        h = h * mk[:, None] * gnorm
        h = _bn(h, bn["g"], bn["b"])

    h = jax.nn.leaky_relu(h @ params["lin1"]["w"] + params["lin1"]["b"])
    h = h * m3[:, None]
    h = jax.nn.leaky_relu(h @ params["lin2"]["w"] + params["lin2"]["b"])
    hf = (h * m3[:, None])[:, 0]

    onehot = (batch[:, None] == jnp.arange(_G)[None, :])
    gmax = jnp.max(jnp.where(onehot, hf[:, None], -jnp.inf), axis=0)
    gmin = jnp.min(jnp.where(onehot, hf[:, None], jnp.inf), axis=0)
    bmax = gmax[batch]
    bmin = gmin[batch]
    probs = (hf - bmin) / (bmax + 1e-6 - bmin)

    ohf = onehot.astype(jnp.float32)
    s = probs @ ohf
    ss = (probs * probs) @ ohf
    pairwise = (s * s) / 2.0
    selfs = ss

    noloop = (row != col).astype(jnp.float32)
    contrib = noloop * weights[row] * probs[row] * probs[col]
    ewg = jax.ops.segment_sum(contrib, batch[row], num_segments=_G) / 2.0
    ecw = pairwise - selfs
    ed = ecw - ewg
    loss = 0.25 * ed * 0.5 - 0.5 * ewg
    return (probs, loss, ewg.mean(), ecw.mean(), ed.mean(), loss.mean())


# speedup vs baseline: 18.0191x; 17.0539x over previous
import functools

import jax
import jax.numpy as jnp
from jax import lax
from jax.experimental import pallas as pl
from jax.experimental.pallas import tpu as pltpu
from jax.experimental.pallas import tpu_sc as plsc

_N = 50000
_E = 800000
_G = 16
_H = 64

_NACC = 50048
_ROWS_PER_TILE = _NACC // 16
_EPAD = 819200
_IDX_ROWS = _EPAD // 128
_IDX_ROWS_PER_TILE = _IDX_ROWS // 16
_GROUPS = _IDX_ROWS_PER_TILE // 8


def _agg64_body(h0, h1, h2, h3, row2d, col2d, zeros, out0, out1, out2, out3,
                rbuf, cbuf, rows_v, acc, semg, sems):
    c = lax.axis_index("c")
    s = lax.axis_index("s")

    def do_quarter(h_hbm, out_hbm):
        sl = pl.ds(s * _ROWS_PER_TILE, _ROWS_PER_TILE)
        pltpu.sync_copy(zeros.at[sl], acc.at[sl])
        plsc.subcore_barrier()

        def body(g, carry):
            r0 = s * _IDX_ROWS_PER_TILE + g * 8
            pltpu.sync_copy(row2d.at[pl.ds(r0, 8)], rbuf)
            pltpu.sync_copy(col2d.at[pl.ds(r0, 8)], cbuf)
            gh = [pltpu.async_copy(h_hbm.at[rbuf.at[j]],
                                   rows_v.at[pl.ds(j * 128, 128)], semg)
                  for j in range(8)]
            for h in gh:
                h.wait()
            sh = [pltpu.async_copy(rows_v.at[pl.ds(j * 128, 128)],
                                   acc.at[cbuf.at[j]], sems, add=True)
                  for j in range(8)]
            for h in sh:
                h.wait()
            return carry

        lax.fori_loop(0, _GROUPS, body, 0)
        plsc.subcore_barrier()
        pltpu.sync_copy(acc.at[sl], out_hbm.at[sl])

    @pl.when(c == 0)
    def _():
        do_quarter(h0, out0)
        do_quarter(h1, out1)

    @pl.when(c == 1)
    def _():
        do_quarter(h2, out2)
        do_quarter(h3, out3)


@jax.jit
def _sc_agg64(h0, h1, h2, h3, row2d, col2d, zeros):
    mesh = plsc.VectorSubcoreMesh(core_axis_name="c", subcore_axis_name="s")
    k = functools.partial(
        pl.kernel,
        mesh=mesh,
        out_type=tuple(jax.ShapeDtypeStruct((_NACC, 16), jnp.float32)
                       for _ in range(4)),
        scratch_types=[
            pltpu.VMEM((8, 128), jnp.int32),
            pltpu.VMEM((8, 128), jnp.int32),
            pltpu.VMEM((1024, 16), jnp.float32),
            pltpu.VMEM_SHARED((_NACC, 16), jnp.float32),
            pltpu.SemaphoreType.DMA,
            pltpu.SemaphoreType.DMA,
        ],
        compiler_params=pltpu.CompilerParams(use_tc_tiling_on_sc=False),
    )(_agg64_body)
    return k(h0, h1, h2, h3, row2d, col2d, zeros)


def _seg_sum_feat(h, row2d, col2d, zeros):
    outs = _sc_agg64(h[:, :16], h[:, 16:32], h[:, 32:48], h[:, 48:],
                     row2d, col2d, zeros)
    return jnp.concatenate([o[:_N] for o in outs], axis=1)


def _seg2_body(srcA, srcB, row2d, col2d, zeros1, outA, outB,
               rbuf, cbuf, gv, acc, semg, sems):
    c = lax.axis_index("c")
    s = lax.axis_index("s")
    sl = pl.ds(s * _ROWS_PER_TILE, _ROWS_PER_TILE)

    def run(src, out_hbm):
        pltpu.sync_copy(zeros1.at[sl], acc.at[sl])
        plsc.subcore_barrier()

        def body(g, carry):
            r0 = s * _IDX_ROWS_PER_TILE + g * 8
            pltpu.sync_copy(row2d.at[pl.ds(r0, 8)], rbuf)
            pltpu.sync_copy(col2d.at[pl.ds(r0, 8)], cbuf)
            gh = [pltpu.async_copy(src.at[rbuf.at[j]],
                                   gv.at[pl.ds(j * 128, 128)], semg)
                  for j in range(8)]
            for h in gh:
                h.wait()
            sh = [pltpu.async_copy(gv.at[pl.ds(j * 128, 128)],
                                   acc.at[cbuf.at[j]], sems, add=True)
                  for j in range(8)]
            for h in sh:
                h.wait()
            return carry
        lax.fori_loop(0, _GROUPS, body, 0)
        plsc.subcore_barrier()
        pltpu.sync_copy(acc.at[sl], out_hbm.at[sl])

    @pl.when(c == 0)
    def _():
        run(srcA, outA)

    @pl.when(c == 1)
    def _():
        run(srcB, outB)


@jax.jit
def _sc_seg2(srcA, srcB, row2d, col2d, zeros1):
    mesh = plsc.VectorSubcoreMesh(core_axis_name="c", subcore_axis_name="s")
    k = functools.partial(
        pl.kernel,
        mesh=mesh,
        out_type=(jax.ShapeDtypeStruct((_NACC,), jnp.float32),
                  jax.ShapeDtypeStruct((_NACC,), jnp.float32)),
        scratch_types=[
            pltpu.VMEM((8, 128), jnp.int32),
            pltpu.VMEM((8, 128), jnp.int32),
            pltpu.VMEM((1024,), jnp.float32),
            pltpu.VMEM_SHARED((_NACC,), jnp.float32),
            pltpu.SemaphoreType.DMA,
            pltpu.SemaphoreType.DMA,
        ],
        compiler_params=pltpu.CompilerParams(use_tc_tiling_on_sc=False),
    )(_seg2_body)
    return k(srcA, srcB, row2d, col2d, zeros1)


_CGROUPS = _IDX_ROWS // 32 // 8


def _contrib_body(aw_hbm, probs_hbm, row2d, col2d, zeros1, o0, o1,
                  rbuf, cbuf, awv, pcv, cvals, acc, semg, sems):
    c = lax.axis_index("c")
    s = lax.axis_index("s")
    wid = c * 16 + s
    sl = pl.ds(s * _ROWS_PER_TILE, _ROWS_PER_TILE)
    pltpu.sync_copy(zeros1.at[sl], acc.at[sl])
    plsc.subcore_barrier()

    def body(g, carry):
        r0 = wid * (_CGROUPS * 8) + g * 8
        pltpu.sync_copy(row2d.at[pl.ds(r0, 8)], rbuf)
        pltpu.sync_copy(col2d.at[pl.ds(r0, 8)], cbuf)
        gh = []
        for j in range(8):
            gh.append(pltpu.async_copy(aw_hbm.at[rbuf.at[j]], awv.at[j], semg))
            gh.append(pltpu.async_copy(probs_hbm.at[cbuf.at[j]], pcv.at[j], semg))
        for h in gh:
            h.wait()
        for j in range(8):
            for kk in range(8):
                di = pl.ds(kk * 16, 16)
                contrib = awv[j, di] * pcv[j, di]
                msk = rbuf[j, di] != cbuf[j, di]
                cvals[j, di] = jnp.where(msk, contrib, 0.0)
        sh = [pltpu.async_copy(cvals.at[j], acc.at[rbuf.at[j]], sems, add=True)
              for j in range(8)]
        for h in sh:
            h.wait()
        return carry

    lax.fori_loop(0, _CGROUPS, body, 0)
    plsc.subcore_barrier()

    @pl.when(c == 0)
    def _():
        pltpu.sync_copy(acc.at[sl], o0.at[sl])

    @pl.when(c == 1)
    def _():
        pltpu.sync_copy(acc.at[sl], o1.at[sl])


@jax.jit
def _sc_contrib(aw, probs_pad, row2d, col2d, zeros1):
    mesh = plsc.VectorSubcoreMesh(core_axis_name="c", subcore_axis_name="s")
    k = functools.partial(
        pl.kernel,
        mesh=mesh,
        out_type=(jax.ShapeDtypeStruct((_NACC,), jnp.float32),
                  jax.ShapeDtypeStruct((_NACC,), jnp.float32)),
        scratch_types=[
            pltpu.VMEM((8, 128), jnp.int32),
            pltpu.VMEM((8, 128), jnp.int32),
            pltpu.VMEM((8, 128), jnp.float32),
            pltpu.VMEM((8, 128), jnp.float32),
            pltpu.VMEM((8, 128), jnp.float32),
            pltpu.VMEM_SHARED((_NACC,), jnp.float32),
            pltpu.SemaphoreType.DMA,
            pltpu.SemaphoreType.DMA,
        ],
        compiler_params=pltpu.CompilerParams(use_tc_tiling_on_sc=False,
                                             needs_layout_passes=False),
    )(_contrib_body)
    return k(aw, probs_pad, row2d, col2d, zeros1)


def _bn(h, g, b):
    mean = jnp.mean(h, axis=0)
    var = jnp.var(h, axis=0)
    return (h - mean) / jnp.sqrt(var + 1e-5) * g + b


def kernel(x, weights, params, edge_index, batch):
    row, col = edge_index[0], edge_index[1]
    gnorm = float(_N) ** -0.5

    pad = _EPAD - _E
    row_p = jnp.concatenate([row, jnp.zeros((pad,), jnp.int32)])
    col_p = jnp.concatenate(
        [col, _N + (jnp.arange(pad, dtype=jnp.int32) % (_NACC - _N))])
    row2d = row_p.reshape(_IDX_ROWS, 128)
    col2d = col_p.reshape(_IDX_ROWS, 128)
    zeros = jnp.zeros((_NACC, 16), jnp.float32)
    zeros1 = jnp.zeros((_NACC,), jnp.float32)
    x_pad = jnp.concatenate([x, jnp.zeros((_NACC - _N,), jnp.float32)])

    ind0 = jnp.where(jnp.abs(x_pad) > 0.0, 1.0, 0.0)
    s_ind0, agg1p = _sc_seg2(ind0, x_pad, row2d, col2d, zeros1)
    m1p = jnp.minimum(ind0 + s_ind0, 1.0)
    s_m1, s_m1b = _sc_seg2(m1p, m1p, row2d, col2d, zeros1)
    m2p = jnp.minimum(m1p + s_m1, 1.0)
    s_m2, s_m2b = _sc_seg2(m2p, m2p, row2d, col2d, zeros1)
    m3p = jnp.minimum(m2p + s_m2, 1.0)
    m1, m2, m3 = m1p[:_N], m2p[:_N], m3p[:_N]
    h = x[:, None]
    p = params["conv1"]
    agg = agg1p[:_N, None]
    z = (1.0 + p["eps"]) * h + agg
    z = jax.nn.relu(z @ p["w1"] + p["b1"])
    z = jax.nn.relu(z @ p["w2"] + p["b2"])
    h = jax.nn.leaky_relu(_bn(z, p["bn_g"], p["bn_b"]))
    h = h * m1[:, None] * gnorm
    h = _bn(h, params["bn1"]["g"], params["bn1"]["b"])

    for p, bn, mk in zip(params["convs"], params["bns"], (m2, m3)):
        agg = _seg_sum_feat(h, row2d, col2d, zeros)
        z = (1.0 + p["eps"]) * h + agg
        z = jax.nn.relu(z @ p["w1"] + p["b1"])
        z = jax.nn.relu(z @ p["w2"] + p["b2"])
        z = _bn(z, p["bn_g"], p["bn_b"])
        h = h + jax.nn.leaky_relu(z)
        h = h * mk[:, None] * gnorm
        h = _bn(h, bn["g"], bn["b"])

    h = jax.nn.leaky_relu(h @ params["lin1"]["w"] + params["lin1"]["b"])
    h = h * m3[:, None]
    h = jax.nn.leaky_relu(h @ params["lin2"]["w"] + params["lin2"]["b"])
    hf = (h * m3[:, None])[:, 0]

    onehot = (batch[:, None] == jnp.arange(_G)[None, :])
    gmax = jnp.max(jnp.where(onehot, hf[:, None], -jnp.inf), axis=0)
    gmin = jnp.min(jnp.where(onehot, hf[:, None], jnp.inf), axis=0)
    bmax = gmax[batch]
    bmin = gmin[batch]
    probs = (hf - bmin) / (bmax + 1e-6 - bmin)

    ohf = onehot.astype(jnp.float32)
    s = probs @ ohf
    ss = (probs * probs) @ ohf
    pairwise = (s * s) / 2.0
    selfs = ss

    zpad = jnp.zeros((_NACC - _N,), jnp.float32)
    aw = jnp.concatenate([weights * probs, zpad])
    probs_pad = jnp.concatenate([probs, zpad])
    e0, e1 = _sc_contrib(aw, probs_pad, row2d, col2d, zeros1)
    ewn = e0[:_N] + e1[:_N]
    ewg = (ewn @ ohf) / 2.0
    ecw = pairwise - selfs
    ed = ecw - ewg
    loss = 0.25 * ed * 0.5 - 0.5 * ewg
    return (probs, loss, ewg.mean(), ecw.mean(), ed.mean(), loss.mean())
